# baseline (speedup 1.0000x reference)
import functools

import jax
import jax.numpy as jnp
from jax import lax
from jax.experimental import pallas as pl
from jax.experimental.pallas import tpu as pltpu
from jax.experimental.pallas import tpu_sc as plsc

_PC = 2380
_NE = (_PC + 1) ** 2
_HW = 225
_F = 32
_G = 4


@functools.cache
def _make_sc_lookup(B):
    info = plsc.get_sparse_core_info()
    num_workers = info.num_cores * info.num_subcores
    nb = B // num_workers
    ng = nb // _G
    mesh = plsc.VectorSubcoreMesh(core_axis_name="c", subcore_axis_name="s")

    @functools.partial(
        pl.kernel,
        mesh=mesh,
        out_type=jax.ShapeDtypeStruct((B * _F * _HW,), jnp.float32),
        scratch_types=[
            pltpu.VMEM((_G * 2 * _HW + 16,), jnp.int32),
            pltpu.VMEM((_G * 2 * _HW + 16,), jnp.float32),
            pltpu.VMEM((240,), jnp.int32),
            pltpu.VMEM((240, _F), jnp.float32),
            pltpu.VMEM((_F * _HW + 16,), jnp.float32),
            pltpu.SemaphoreType.DMA,
            pltpu.SemaphoreType.DMA,
        ],
        compiler_params=pltpu.CompilerParams(
            needs_layout_passes=False, use_tc_tiling_on_sc=False),
    )
    def k(sp_hbm, bo_hbm, tab_hbm, out_hbm, spv, bov, idxv, rows, outv, sem0, sem1):
        wid = lax.axis_index("s") * info.num_cores + lax.axis_index("c")
        iota16 = lax.iota(jnp.int32, 16)
        spv[pl.ds(_G * 2 * _HW, 16)] = jnp.zeros((16,), jnp.int32)
        bov[pl.ds(_G * 2 * _HW, 16)] = jnp.zeros((16,), jnp.float32)

        def body(g, carry):
            gb = wid * nb + g * _G
            pltpu.sync_copy(sp_hbm.at[pl.ds(gb * 2 * _HW, _G * 2 * _HW)],
                            spv.at[pl.ds(0, _G * 2 * _HW)])
            pltpu.sync_copy(bo_hbm.at[pl.ds(gb * 2 * _HW, _G * 2 * _HW)],
                            bov.at[pl.ds(0, _G * 2 * _HW)])
            for j in range(_G):
                jo = j * 2 * _HW
                for c in range(15):
                    p0 = spv[pl.ds(jo + c * 16, 16)]
                    p1 = spv[pl.ds(jo + _HW + c * 16, 16)]
                    occ = (bov[pl.ds(jo + c * 16, 16)]
                           + bov[pl.ds(jo + _HW + c * 16, 16)]) > 0.0
                    idxv[pl.ds(c * 16, 16)] = jnp.where(
                        occ, jnp.int32(_NE - 1), p1 * (_PC + 1) + p0)
                g0 = pltpu.async_copy(tab_hbm.at[idxv.at[pl.ds(0, 120)]],
                                      rows.at[pl.ds(0, 120)], sem0)
                g1 = pltpu.async_copy(tab_hbm.at[idxv.at[pl.ds(120, 105)]],
                                      rows.at[pl.ds(120, 105)], sem1)
                g0.wait()
                g1.wait()

                def tbody(cell, tc):
                    v0 = rows[cell, pl.ds(0, 16)]
                    v1 = rows[cell, pl.ds(16, 16)]
                    a0 = iota16 * _HW + cell
                    plsc.store_scatter(outv, [a0], v0)
                    plsc.store_scatter(outv, [a0 + 16 * _HW], v1)
                    return tc
                lax.fori_loop(0, _HW, tbody, 0)
                pltpu.sync_copy(outv.at[pl.ds(0, _F * _HW)],
                                out_hbm.at[pl.ds((gb + j) * _F * _HW, _F * _HW)])
            return carry
        lax.fori_loop(0, ng, body, 0)

    return k


def kernel(sparse_feature_input, sparse_feature_dim, board_input, pcode_embedding):
    B = sparse_feature_input.shape[0]
    sp = sparse_feature_input[:, 10:12].reshape(B * 2 * _HW)
    bo = board_input.reshape(B * 2 * _HW)
    out = _make_sc_lookup(B)(sp, bo, pcode_embedding)
    return out.reshape(B, _F, 15, 15)


# speedup vs baseline: 1.3470x; 1.3470x over previous
import functools

import jax
import jax.numpy as jnp
from jax import lax
from jax.experimental import pallas as pl
from jax.experimental.pallas import tpu as pltpu
from jax.experimental.pallas import tpu_sc as plsc

_PC = 2380
_NE = (_PC + 1) ** 2
_HW = 225
_F = 32
_G = 4
_BB = 128


@functools.cache
def _make_sc_lookup(B):
    info = plsc.get_sparse_core_info()
    num_workers = info.num_cores * info.num_subcores
    assert B == num_workers * _BB
    ng = _BB // _G
    mesh = plsc.VectorSubcoreMesh(core_axis_name="c", subcore_axis_name="s")

    @functools.partial(
        pl.kernel,
        mesh=mesh,
        out_type=jax.ShapeDtypeStruct((B * _F * _HW,), jnp.float32),
        scratch_types=[
            pltpu.VMEM((_G * 2 * _HW + 16,), jnp.int32),
            pltpu.VMEM((_G * 2 * _HW + 16,), jnp.float32),
            pltpu.VMEM((240 * _BB,), jnp.int32),
            pltpu.VMEM((_BB, _F), jnp.float32),
            pltpu.VMEM((_BB, _F), jnp.float32),
            pltpu.VMEM((_F * _BB,), jnp.float32),
            pltpu.VMEM((_F * _BB,), jnp.float32),
            pltpu.SemaphoreType.DMA,
            pltpu.SemaphoreType.DMA,
            pltpu.SemaphoreType.DMA,
            pltpu.SemaphoreType.DMA,
        ],
        compiler_params=pltpu.CompilerParams(
            needs_layout_passes=False, use_tc_tiling_on_sc=False),
    )
    def k(sp_hbm, bo_hbm, tab_hbm, out_hbm, spv, bov, idxT, rows0, rows1,
          blk0, blk1, gsem0, gsem1, osem0, osem1):
        wid = lax.axis_index("s") * info.num_cores + lax.axis_index("c")
        iota16 = lax.iota(jnp.int32, 16)
        a_lo = (iota16 // 8) * 1024 + (iota16 % 8) * 128
        a_hi = a_lo + 2048

        def p1body(g, carry):
            gb = (wid * _BB + g * _G) * 2 * _HW
            pltpu.sync_copy(sp_hbm.at[pl.ds(gb, _G * 2 * _HW)],
                            spv.at[pl.ds(0, _G * 2 * _HW)])
            pltpu.sync_copy(bo_hbm.at[pl.ds(gb, _G * 2 * _HW)],
                            bov.at[pl.ds(0, _G * 2 * _HW)])
            for j in range(_G):
                jo = j * 2 * _HW
                bl = g * _G + j
                for c in range(15):
                    p0 = spv[pl.ds(jo + c * 16, 16)]
                    p1 = spv[pl.ds(jo + _HW + c * 16, 16)]
                    occ = (bov[pl.ds(jo + c * 16, 16)]
                           + bov[pl.ds(jo + _HW + c * 16, 16)]) > 0.0
                    idx = jnp.where(occ, jnp.int32(_NE - 1), p1 * (_PC + 1) + p0)
                    plsc.store_scatter(idxT, [iota16 * _BB + (c * 16 * _BB + bl)], idx)
            return carry
        lax.fori_loop(0, ng, p1body, 0)

        def gather(hw, rbuf, sem):
            return pltpu.async_copy(tab_hbm.at[idxT.at[pl.ds(hw * _BB, _BB)]],
                                    rbuf, sem)

        def transpose(rbuf, bbuf):
            def tb(q, carry):
                for u in range(4):
                    b = q * 4 + u
                    v0 = rbuf[b, pl.ds(0, 16)]
                    v1 = rbuf[b, pl.ds(16, 16)]
                    plsc.store_scatter(bbuf, [a_lo + b], v0)
                    plsc.store_scatter(bbuf, [a_hi + b], v1)
                return carry
            lax.fori_loop(0, _BB // 4, tb, 0)

        def out_base(hw):
            return (hw * 4 * 32 + wid) * 1024

        def store_out(hw, bbuf, sem):
            for fb in range(4):
                pltpu.async_copy(
                    bbuf.at[pl.ds(fb * 1024, 1024)],
                    out_hbm.at[pl.ds(out_base(hw) + fb * 32 * 1024, 1024)], sem)

        def wait_out(hw, bbuf, sem):
            for fb in range(4):
                pltpu.make_async_copy(
                    bbuf.at[pl.ds(fb * 1024, 1024)],
                    out_hbm.at[pl.ds(out_base(hw) + fb * 32 * 1024, 1024)],
                    sem).wait()

        gather(0, rows0, gsem0)

        def p2body(t, carry):
            hw0 = 2 * t
            hw1 = 2 * t + 1
            gather(hw1, rows1, gsem1)
            pltpu.make_async_copy(tab_hbm.at[idxT.at[pl.ds(hw0 * _BB, _BB)]],
                                  rows0, gsem0).wait()

            @pl.when(t > 0)
            def _():
                wait_out(2 * t - 2, blk0, osem0)
            transpose(rows0, blk0)
            store_out(hw0, blk0, osem0)
            gather(hw0 + 2, rows0, gsem0)

            pltpu.make_async_copy(tab_hbm.at[idxT.at[pl.ds(hw1 * _BB, _BB)]],
                                  rows1, gsem1).wait()

            @pl.when(t > 0)
            def _():
                wait_out(2 * t - 1, blk1, osem1)
            transpose(rows1, blk1)
            store_out(hw1, blk1, osem1)
            return carry
        lax.fori_loop(0, (_HW - 1) // 2, p2body, 0)

        last = _HW - 1
        pltpu.make_async_copy(tab_hbm.at[idxT.at[pl.ds(last * _BB, _BB)]],
                              rows0, gsem0).wait()
        wait_out(last - 2, blk0, osem0)
        transpose(rows0, blk0)
        store_out(last, blk0, osem0)
        wait_out(last - 1, blk1, osem1)
        wait_out(last, blk0, osem0)

    return k


def kernel(sparse_feature_input, sparse_feature_dim, board_input, pcode_embedding):
    B = sparse_feature_input.shape[0]
    sp = sparse_feature_input[:, 10:12].reshape(B * 2 * _HW)
    bo = board_input.reshape(B * 2 * _HW)
    out = _make_sc_lookup(B)(sp, bo, pcode_embedding)
    y = out.reshape(15, 15, 4, 32, 8, 128)
    y = y.transpose(3, 5, 2, 4, 0, 1)
    return y.reshape(B, _F, 15, 15)


# speedup vs baseline: 1.3615x; 1.0108x over previous
import functools

import jax
import jax.numpy as jnp
from jax import lax
from jax.experimental import pallas as pl
from jax.experimental.pallas import tpu as pltpu
from jax.experimental.pallas import tpu_sc as plsc

_PC = 2380
_NE = (_PC + 1) ** 2
_HW = 225
_F = 32
_G = 4
_BB = 128
_GW = _G * 2 * _HW


@functools.cache
def _make_sc_lookup(B):
    info = plsc.get_sparse_core_info()
    num_workers = info.num_cores * info.num_subcores
    assert B == num_workers * _BB
    ng = _BB // _G
    mesh = plsc.VectorSubcoreMesh(core_axis_name="c", subcore_axis_name="s")

    @functools.partial(
        pl.kernel,
        mesh=mesh,
        out_type=jax.ShapeDtypeStruct((B * _F * _HW,), jnp.float32),
        scratch_types=[
            pltpu.VMEM((2, _GW + 16), jnp.int32),
            pltpu.VMEM((2, _GW + 16), jnp.float32),
            pltpu.VMEM((240 * _BB,), jnp.int32),
            pltpu.VMEM((4, _BB, _F), jnp.float32),
            pltpu.VMEM((2, _F * _BB), jnp.float32),
            pltpu.SemaphoreType.DMA,
            pltpu.SemaphoreType.DMA,
            pltpu.SemaphoreType.DMA,
            pltpu.SemaphoreType.DMA,
            pltpu.SemaphoreType.DMA,
            pltpu.SemaphoreType.DMA,
            pltpu.SemaphoreType.DMA,
            pltpu.SemaphoreType.DMA,
        ],
        compiler_params=pltpu.CompilerParams(
            needs_layout_passes=False, use_tc_tiling_on_sc=False),
    )
    def k(sp_hbm, bo_hbm, tab_hbm, out_hbm, spv, bov, idxT, rows, blk,
          isem0, isem1, gsem0, gsem1, gsem2, gsem3, osem0, osem1):
        wid = lax.axis_index("s") * info.num_cores + lax.axis_index("c")
        iota16 = lax.iota(jnp.int32, 16)
        gsems = (gsem0, gsem1, gsem2, gsem3)
        isems = (isem0, isem1)
        osems = (osem0, osem1)
        a_lo = (iota16 // 8) * 1024 + (iota16 % 8) * 128
        a_hi = a_lo + 2048

        def in_copy(g, p):
            gb = (wid * _BB + g * _G) * 2 * _HW
            sp_c = pltpu.async_copy(sp_hbm.at[pl.ds(gb, _GW)],
                                    spv.at[p, pl.ds(0, _GW)], isems[p])
            bo_c = pltpu.async_copy(bo_hbm.at[pl.ds(gb, _GW)],
                                    bov.at[p, pl.ds(0, _GW)], isems[p])
            return sp_c, bo_c

        def in_wait(g, p):
            gb = (wid * _BB + g * _G) * 2 * _HW
            pltpu.make_async_copy(sp_hbm.at[pl.ds(gb, _GW)],
                                  spv.at[p, pl.ds(0, _GW)], isems[p]).wait()
            pltpu.make_async_copy(bo_hbm.at[pl.ds(gb, _GW)],
                                  bov.at[p, pl.ds(0, _GW)], isems[p]).wait()

        def compute_group(g, p):
            for j in range(_G):
                jo = j * 2 * _HW
                bl = g * _G + j
                for c in range(15):
                    p0 = spv[p, pl.ds(jo + c * 16, 16)]
                    p1 = spv[p, pl.ds(jo + _HW + c * 16, 16)]
                    occ = (bov[p, pl.ds(jo + c * 16, 16)]
                           + bov[p, pl.ds(jo + _HW + c * 16, 16)]) > 0.0
                    idx = jnp.where(occ, jnp.int32(_NE - 1), p1 * (_PC + 1) + p0)
                    plsc.store_scatter(idxT, [iota16 * _BB + (c * 16 * _BB + bl)], idx)

        in_copy(0, 0)

        def p1body(h, carry):
            g0 = 2 * h

            @pl.when(g0 + 1 < ng)
            def _():
                in_copy(g0 + 1, 1)
            in_wait(g0, 0)
            compute_group(g0, 0)

            @pl.when(g0 + 2 < ng)
            def _():
                in_copy(g0 + 2, 0)
            in_wait(g0 + 1, 1)
            compute_group(g0 + 1, 1)
            return carry
        lax.fori_loop(0, ng // 2, p1body, 0)

        def gather(hw, u):
            pltpu.async_copy(tab_hbm.at[idxT.at[pl.ds(hw * _BB, _BB)]],
                             rows.at[u], gsems[u])

        def gather_wait(hw, u):
            pltpu.make_async_copy(tab_hbm.at[idxT.at[pl.ds(hw * _BB, _BB)]],
                                  rows.at[u], gsems[u]).wait()

        def transpose(u, p):
            def tb(q, carry):
                for v in range(4):
                    b = q * 4 + v
                    v0 = rows[u, b, pl.ds(0, 16)]
                    v1 = rows[u, b, pl.ds(16, 16)]
                    plsc.store_scatter(blk.at[p], [a_lo + b], v0)
                    plsc.store_scatter(blk.at[p], [a_hi + b], v1)
                return carry
            lax.fori_loop(0, _BB // 4, tb, 0)

        def store_out(hw, p):
            for fb in range(4):
                pltpu.async_copy(
                    blk.at[p, pl.ds(fb * 1024, 1024)],
                    out_hbm.at[pl.ds((hw * 128 + fb * 32 + wid) * 1024, 1024)],
                    osems[p])

        def wait_out(hw, p):
            for fb in range(4):
                pltpu.make_async_copy(
                    blk.at[p, pl.ds(fb * 1024, 1024)],
                    out_hbm.at[pl.ds((hw * 128 + fb * 32 + wid) * 1024, 1024)],
                    osems[p]).wait()

        def cell(hw, u, p):
            gather_wait(hw, u)

            @pl.when(hw >= 2)
            def _():
                wait_out(hw - 2, p)
            transpose(u, p)
            store_out(hw, p)

        gather(0, 0)
        gather(1, 1)
        gather(2, 2)

        def p2body(t, carry):
            c0 = 4 * t
            for v in range(4):
                hw = c0 + v
                nxt = hw + 3

                @pl.when(nxt < _HW)
                def _():
                    gather(nxt, (v + 3) % 4)
                cell(hw, v, v % 2)
            return carry
        lax.fori_loop(0, _HW // 4, p2body, 0)

        last = _HW - 1
        gather_wait(last, 0)
        wait_out(last - 2, 0)
        transpose(0, 0)
        store_out(last, 0)
        wait_out(last - 1, 1)
        wait_out(last, 0)

    return k


def kernel(sparse_feature_input, sparse_feature_dim, board_input, pcode_embedding):
    B = sparse_feature_input.shape[0]
    sp = sparse_feature_input[:, 10:12].reshape(B * 2 * _HW)
    bo = board_input.reshape(B * 2 * _HW)
    out = _make_sc_lookup(B)(sp, bo, pcode_embedding)
    y = out.reshape(15, 15, 4, 32, 8, 128)
    y = y.transpose(3, 5, 2, 4, 0, 1)
    return y.reshape(B, _F, 15, 15)


# speedup vs baseline: 1.5355x; 1.1278x over previous
import functools

import jax
import jax.numpy as jnp
from jax import lax
from jax.experimental import pallas as pl
from jax.experimental.pallas import tpu as pltpu
from jax.experimental.pallas import tpu_sc as plsc

_PC = 2380
_NE = (_PC + 1) ** 2
_HW = 225
_F = 32
_G = 4
_BB = 128
_GW = _G * 2 * _HW


@functools.cache
def _make_sc_lookup(B):
    info = plsc.get_sparse_core_info()
    num_workers = info.num_cores * info.num_subcores
    assert B == num_workers * _BB
    ng = _BB // _G
    mesh = plsc.VectorSubcoreMesh(core_axis_name="c", subcore_axis_name="s")

    @functools.partial(
        pl.kernel,
        mesh=mesh,
        out_type=jax.ShapeDtypeStruct((B * _F * _HW,), jnp.float32),
        scratch_types=[
            pltpu.VMEM((2, _GW + 16), jnp.int32),
            pltpu.VMEM((2, _GW + 16), jnp.float32),
            pltpu.VMEM((240 * _BB,), jnp.int32),
            pltpu.VMEM((4, _BB, _F), jnp.float32),
            pltpu.VMEM((2, _F * 136 + 8), jnp.float32),
            pltpu.SemaphoreType.DMA,
            pltpu.SemaphoreType.DMA,
            pltpu.SemaphoreType.DMA,
            pltpu.SemaphoreType.DMA,
            pltpu.SemaphoreType.DMA,
            pltpu.SemaphoreType.DMA,
            pltpu.SemaphoreType.DMA,
            pltpu.SemaphoreType.DMA,
        ],
        compiler_params=pltpu.CompilerParams(
            needs_layout_passes=False, use_tc_tiling_on_sc=False),
    )
    def k(sp_hbm, bo_hbm, tab_hbm, out_hbm, spv, bov, idxT, rows, blk,
          isem0, isem1, gsem0, gsem1, gsem2, gsem3, osem0, osem1):
        wid = lax.axis_index("s") * info.num_cores + lax.axis_index("c")
        iota16 = lax.iota(jnp.int32, 16)
        gsems = (gsem0, gsem1, gsem2, gsem3)
        isems = (isem0, isem1)
        osems = (osem0, osem1)
        a_lo = iota16 * 136
        a_hi = (iota16 + 16) * 136

        def in_copy(g, p):
            gb = (wid * _BB + g * _G) * 2 * _HW
            sp_c = pltpu.async_copy(sp_hbm.at[pl.ds(gb, _GW)],
                                    spv.at[p, pl.ds(0, _GW)], isems[p])
            bo_c = pltpu.async_copy(bo_hbm.at[pl.ds(gb, _GW)],
                                    bov.at[p, pl.ds(0, _GW)], isems[p])
            return sp_c, bo_c

        def in_wait(g, p):
            gb = (wid * _BB + g * _G) * 2 * _HW
            pltpu.make_async_copy(sp_hbm.at[pl.ds(gb, _GW)],
                                  spv.at[p, pl.ds(0, _GW)], isems[p]).wait()
            pltpu.make_async_copy(bo_hbm.at[pl.ds(gb, _GW)],
                                  bov.at[p, pl.ds(0, _GW)], isems[p]).wait()

        def compute_group(g, p):
            for j in range(_G):
                jo = j * 2 * _HW
                bl = g * _G + j
                for c in range(15):
                    p0 = spv[p, pl.ds(jo + c * 16, 16)]
                    p1 = spv[p, pl.ds(jo + _HW + c * 16, 16)]
                    occ = (bov[p, pl.ds(jo + c * 16, 16)]
                           + bov[p, pl.ds(jo + _HW + c * 16, 16)]) > 0.0
                    idx = jnp.where(occ, jnp.int32(_NE - 1), p1 * (_PC + 1) + p0)
                    plsc.store_scatter(idxT, [iota16 * _BB + (c * 16 * _BB + bl)], idx)

        in_copy(0, 0)

        def p1body(h, carry):
            g0 = 2 * h

            @pl.when(g0 + 1 < ng)
            def _():
                in_copy(g0 + 1, 1)
            in_wait(g0, 0)
            compute_group(g0, 0)

            @pl.when(g0 + 2 < ng)
            def _():
                in_copy(g0 + 2, 0)
            in_wait(g0 + 1, 1)
            compute_group(g0 + 1, 1)
            return carry
        with jax.named_scope("phase1_idx"):
            lax.fori_loop(0, ng // 2, p1body, 0)

        def gather(hw, u):
            pltpu.async_copy(tab_hbm.at[idxT.at[pl.ds(hw * _BB, _BB)]],
                             rows.at[u], gsems[u])

        def gather_wait(hw, u):
            pltpu.make_async_copy(tab_hbm.at[idxT.at[pl.ds(hw * _BB, _BB)]],
                                  rows.at[u], gsems[u]).wait()

        def transpose(u, p):
            def tb(q, carry):
                for v in range(4):
                    b = q * 4 + v
                    v0 = rows[u, b, pl.ds(0, 16)]
                    v1 = rows[u, b, pl.ds(16, 16)]
                    plsc.store_scatter(blk.at[p], [a_lo + b], v0)
                    plsc.store_scatter(blk.at[p], [a_hi + b], v1)
                return carry
            lax.fori_loop(0, _BB // 4, tb, 0)

        def _out_pairs(hw, p):
            for f in range(_F):
                yield (blk.at[p, pl.ds(f * 136, _BB)],
                       out_hbm.at[pl.ds(
                           (hw * 128 + (f // 8) * 32 + wid) * 1024
                           + (f % 8) * 128, _BB)])

        def store_out(hw, p):
            for src, dst in _out_pairs(hw, p):
                pltpu.async_copy(src, dst, osems[p])

        def wait_out(hw, p):
            for src, dst in _out_pairs(hw, p):
                pltpu.make_async_copy(src, dst, osems[p]).wait()

        def cell(hw, u, p):
            gather_wait(hw, u)

            @pl.when(hw >= 2)
            def _():
                wait_out(hw - 2, p)
            transpose(u, p)
            store_out(hw, p)

        gather(0, 0)
        gather(1, 1)
        gather(2, 2)

        def p2body(t, carry):
            c0 = 4 * t
            for v in range(4):
                hw = c0 + v
                nxt = hw + 3

                @pl.when(nxt < _HW)
                def _():
                    gather(nxt, (v + 3) % 4)
                cell(hw, v, v % 2)
            return carry
        with jax.named_scope("phase2_gather"):
            lax.fori_loop(0, _HW // 4, p2body, 0)

        last = _HW - 1
        gather_wait(last, 0)
        wait_out(last - 2, 0)
        transpose(0, 0)
        store_out(last, 0)
        wait_out(last - 1, 1)
        wait_out(last, 0)

    return k


def kernel(sparse_feature_input, sparse_feature_dim, board_input, pcode_embedding):
    B = sparse_feature_input.shape[0]
    sp = sparse_feature_input[:, 10:12].reshape(B * 2 * _HW)
    bo = board_input.reshape(B * 2 * _HW)
    out = _make_sc_lookup(B)(sp, bo, pcode_embedding)
    y = out.reshape(15, 15, 4, 32, 8, 128)
    y = y.transpose(3, 5, 2, 4, 0, 1)
    return y.reshape(B, _F, 15, 15)
